# Initial kernel scaffold; baseline (speedup 1.0000x reference)
import functools

import jax
import jax.numpy as jnp
from jax import lax
from jax.experimental import pallas as pl
from jax.experimental.pallas import tpu as pltpu
from jax.experimental.pallas import tpu_sc as plsc

N = 10000
E = 160000
D_IN = 256
D_HID = 512
D_OUT = 256
HALF = 128
NC = 2
NS = 16
L = 16
C = 128
EPT = 10240
E_PAD = NS * EPT
NPT = N // NS
N_HIST = 10112
DEG_EPT = E_PAD // (NC * NS)

_mesh = plsc.VectorSubcoreMesh(core_axis_name="c", subcore_axis_name="s")


def _zero_rows(buf, nrows, width):
    zz = jnp.zeros((L,), jnp.float32)

    def body(e, carry):
        for j in range(width // L):
            buf[e, pl.ds(j * L, L)] = zz
        return carry

    lax.fori_loop(0, nrows, body, 0)


def _deg_body(dst_hbm, ew_hbm, deg0_hbm, deg1_hbm,
              hist, id2d, dstv, eww, degsh):
    c = lax.axis_index("c")
    s = lax.axis_index("s")

    zz = jnp.zeros((L,), jnp.float32)

    def zinit(k, carry):
        hist[pl.ds(k * L, L)] = zz
        return carry

    lax.fori_loop(0, N_HIST // L, zinit, 0)

    def iinit(k, carry):
        base = k * L + lax.broadcasted_iota(jnp.int32, (L,), 0)
        j = k // (C // L)
        i = k % (C // L)
        id2d[j, pl.ds(i * L, L)] = base
        return carry

    lax.fori_loop(0, N_HIST // L, iinit, 0)

    nps = N_HIST // NS
    zoff = s * nps
    pltpu.sync_copy(hist.at[pl.ds(0, nps)], degsh.at[pl.ds(zoff, nps)])
    plsc.subcore_barrier()

    def chunk(k, carry):
        base = c * (E_PAD // NC) + s * DEG_EPT + k * C
        pltpu.sync_copy(dst_hbm.at[pl.ds(base, C)], dstv)
        pltpu.sync_copy(ew_hbm.at[pl.ds(base, C)], eww)
        for i in range(C // L):
            idx = dstv[pl.ds(i * L, L)]
            val = eww[pl.ds(i * L, L)]
            plsc.addupdate_scatter(hist, [idx], val)
        return carry

    lax.fori_loop(0, DEG_EPT // C, chunk, 0)

    def comb(j, carry):
        pltpu.sync_copy(hist.at[pl.ds(j * C, C)], degsh.at[id2d.at[j]], add=True)
        return carry

    lax.fori_loop(0, N_HIST // C, comb, 0)
    plsc.subcore_barrier()

    pltpu.sync_copy(degsh.at[pl.ds(zoff, nps)], hist.at[pl.ds(0, nps)])

    @pl.when(c == 0)
    def _():
        pltpu.sync_copy(hist.at[pl.ds(0, nps)], deg0_hbm.at[pl.ds(zoff, nps)])

    @pl.when(c == 1)
    def _():
        pltpu.sync_copy(hist.at[pl.ds(0, nps)], deg1_hbm.at[pl.ds(zoff, nps)])


_deg_kernel = functools.partial(
    pl.kernel,
    out_type=(jax.ShapeDtypeStruct((N_HIST,), jnp.float32),
              jax.ShapeDtypeStruct((N_HIST,), jnp.float32)),
    mesh=_mesh,
    scratch_types=[
        pltpu.VMEM((N_HIST,), jnp.float32),
        pltpu.VMEM((N_HIST // C, C), jnp.int32),
        pltpu.VMEM((C,), jnp.int32),
        pltpu.VMEM((C,), jnp.float32),
        pltpu.VMEM_SHARED((N_HIST,), jnp.float32),
    ],
)(_deg_body)


def _agg_body(y_hbm, src_hbm, dst_hbm, ew_hbm, t_hbm,
              srcv, gidx, dstv, eww, rows, acc, sem):
    c = lax.axis_index("c")
    s = lax.axis_index("s")

    _zero_rows(rows, C, HALF)
    abase = s * NPT
    for off, sz in ((0, C), (C, C), (2 * C, C), (3 * C, C), (4 * C, NPT - 4 * C)):
        pltpu.sync_copy(rows.at[pl.ds(0, sz)], acc.at[pl.ds(abase + off, sz)])
    plsc.subcore_barrier()

    coff = c * N

    def chunk(k, carry):
        base = s * EPT + k * C
        pltpu.sync_copy(src_hbm.at[pl.ds(base, C)], srcv)
        pltpu.sync_copy(dst_hbm.at[pl.ds(base, C)], dstv)
        pltpu.sync_copy(ew_hbm.at[pl.ds(base, C)], eww)
        for i in range(C // L):
            gidx[pl.ds(i * L, L)] = srcv[pl.ds(i * L, L)] + coff
        pltpu.async_copy(y_hbm.at[gidx], rows, sem).wait()

        def scale(e, carry2):
            sp = plsc.load_gather(eww, [jnp.full((L,), e, jnp.int32)])
            for j in range(HALF // L):
                rows[e, pl.ds(j * L, L)] = rows[e, pl.ds(j * L, L)] * sp
            return carry2

        lax.fori_loop(0, C, scale, 0)
        pltpu.sync_copy(rows, acc.at[dstv], add=True)
        return carry

    lax.fori_loop(0, EPT // C, chunk, 0)
    plsc.subcore_barrier()

    for off, sz in ((0, C), (C, C), (2 * C, C), (3 * C, C), (4 * C, NPT - 4 * C)):
        pltpu.sync_copy(acc.at[pl.ds(abase + off, sz)], rows.at[pl.ds(0, sz)])
        pltpu.sync_copy(rows.at[pl.ds(0, sz)],
                        t_hbm.at[pl.ds(coff + abase + off, sz)])


_agg_kernel = functools.partial(
    pl.kernel,
    out_type=jax.ShapeDtypeStruct((2 * N, HALF), jnp.float32),
    mesh=_mesh,
    scratch_types=[
        pltpu.VMEM((C,), jnp.int32),
        pltpu.VMEM((C,), jnp.int32),
        pltpu.VMEM((C,), jnp.int32),
        pltpu.VMEM((C,), jnp.float32),
        pltpu.VMEM((C, HALF), jnp.float32),
        pltpu.VMEM_SHARED((N, HALF), jnp.float32),
        pltpu.SemaphoreType.DMA,
    ],
)(_agg_body)


def _dinv(d0, d1):
    deg = d0 + d1 + 1.0
    return jnp.where(deg > 0, lax.rsqrt(deg), 0.0)


def _prescale_body(x_ref, d0_ref, d1_ref, y_ref):
    y_ref[...] = _dinv(d0_ref[...], d1_ref[...]) * x_ref[...]


def _main_body(t1a_ref, t1b_ref, y1a_ref, y1b_ref, d0_ref, d1_ref,
               w1_ref, b1_ref, w2_ref, y2_ref):
    dinv = _dinv(d0_ref[...], d1_ref[...])
    agg = jnp.concatenate(
        [dinv * (t1a_ref[...] + y1a_ref[...]),
         dinv * (t1b_ref[...] + y1b_ref[...])], axis=1)
    h = jnp.maximum(
        jnp.dot(agg, w1_ref[...], preferred_element_type=jnp.float32)
        + b1_ref[...], 0.0)
    z = jnp.dot(h, w2_ref[...], preferred_element_type=jnp.float32)
    y2_ref[0] = _dinv(d0_ref[...], d1_ref[...]) * z[:, :HALF]
    y2_ref[1] = _dinv(d0_ref[...], d1_ref[...]) * z[:, HALF:]


def _final_body(t2_ref, y2_ref, d0_ref, d1_ref, b2_ref, out_ref):
    dinv = _dinv(d0_ref[...], d1_ref[...])
    out_ref[...] = dinv * (t2_ref[...] + y2_ref[...]) + b2_ref[...]


def kernel(x, edge_index, edge_weight, W1, b1, W2, b2):
    src = edge_index[0].astype(jnp.int32)
    dst = edge_index[1].astype(jnp.int32)
    ew = edge_weight.astype(jnp.float32)
    pad = E_PAD - E
    src_p = jnp.concatenate([src, jnp.zeros((pad,), jnp.int32)])
    dst_p = jnp.concatenate([dst, jnp.zeros((pad,), jnp.int32)])
    ew_p = jnp.concatenate([ew, jnp.zeros((pad,), jnp.float32)])

    deg0, deg1 = _deg_kernel(dst_p, ew_p)
    d0 = deg0[:N].reshape(N, 1)
    d1 = deg1[:N].reshape(N, 1)

    RB = 2000
    nrb = N // RB
    y1 = pl.pallas_call(
        _prescale_body,
        grid=(nrb, 2),
        in_specs=[
            pl.BlockSpec((RB, HALF), lambda i, h: (i, h)),
            pl.BlockSpec((RB, 1), lambda i, h: (i, 0)),
            pl.BlockSpec((RB, 1), lambda i, h: (i, 0)),
        ],
        out_specs=pl.BlockSpec((RB, HALF), lambda i, h: (i + h * nrb, 0)),
        out_shape=jax.ShapeDtypeStruct((2 * N, HALF), jnp.float32),
    )(x, d0, d1)

    t1 = _agg_kernel(y1, src_p, dst_p, ew_p)

    RM = 1000
    nrm = N // RM
    y2_3d = pl.pallas_call(
        _main_body,
        grid=(nrm,),
        in_specs=[
            pl.BlockSpec((RM, HALF), lambda i: (i, 0)),
            pl.BlockSpec((RM, HALF), lambda i: (i + nrm, 0)),
            pl.BlockSpec((RM, HALF), lambda i: (i, 0)),
            pl.BlockSpec((RM, HALF), lambda i: (i + nrm, 0)),
            pl.BlockSpec((RM, 1), lambda i: (i, 0)),
            pl.BlockSpec((RM, 1), lambda i: (i, 0)),
            pl.BlockSpec((D_IN, D_HID), lambda i: (0, 0)),
            pl.BlockSpec((1, D_HID), lambda i: (0, 0)),
            pl.BlockSpec((D_HID, D_OUT), lambda i: (0, 0)),
        ],
        out_specs=pl.BlockSpec((2, RM, HALF), lambda i: (0, i, 0)),
        out_shape=jax.ShapeDtypeStruct((2, N, HALF), jnp.float32),
    )(t1, t1, y1, y1, d0, d1, W1, b1.reshape(1, D_HID), W2)
    y2 = y2_3d.reshape(2 * N, HALF)

    t2 = _agg_kernel(y2, src_p, dst_p, ew_p)

    out = pl.pallas_call(
        _final_body,
        grid=(nrb, 2),
        in_specs=[
            pl.BlockSpec((RB, HALF), lambda i, h: (i + h * nrb, 0)),
            pl.BlockSpec((RB, HALF), lambda i, h: (i + h * nrb, 0)),
            pl.BlockSpec((RB, 1), lambda i, h: (i, 0)),
            pl.BlockSpec((RB, 1), lambda i, h: (i, 0)),
            pl.BlockSpec((1, HALF), lambda i, h: (h, 0)),
        ],
        out_specs=pl.BlockSpec((RB, HALF), lambda i, h: (i, h)),
        out_shape=jax.ShapeDtypeStruct((N, D_OUT), jnp.float32),
    )(t2, y2, d0, d1, b2.reshape(2, HALF))
    return out


# speedup vs baseline: 5.8763x; 5.8763x over previous
import functools

import jax
import jax.numpy as jnp
from jax import lax
from jax.experimental import pallas as pl
from jax.experimental.pallas import tpu as pltpu
from jax.experimental.pallas import tpu_sc as plsc

N = 10000
E = 160000
D_IN = 256
D_HID = 512
D_OUT = 256
HALF = 128
NC = 2
NS = 16
L = 16
C = 128
EPT = 10240
E_PAD = NS * EPT
NPT = N // NS
N_HIST = 10112
DEG_EPT = E_PAD // (NC * NS)

_mesh = plsc.VectorSubcoreMesh(
    core_axis_name="c", subcore_axis_name="s", num_cores=NC, num_subcores=NS)


def _zero_rows(buf, nrows, width):
    zz = jnp.zeros((L,), jnp.float32)

    def body(e, carry):
        for j in range(width // L):
            buf[e, pl.ds(j * L, L)] = zz
        return carry

    lax.fori_loop(0, nrows, body, 0)


def _deg_body(dst_hbm, ew_hbm, deg0_hbm, deg1_hbm,
              hist, id2d, dstv, eww, degsh):
    c = lax.axis_index("c")
    s = lax.axis_index("s")

    zz = jnp.zeros((L,), jnp.float32)

    def zinit(k, carry):
        hist[pl.ds(k * L, L)] = zz
        return carry

    lax.fori_loop(0, N_HIST // L, zinit, 0)

    def iinit(k, carry):
        base = k * L + lax.broadcasted_iota(jnp.int32, (L,), 0)
        j = k // (C // L)
        i = k % (C // L)
        id2d[j, pl.ds(i * L, L)] = base
        return carry

    lax.fori_loop(0, N_HIST // L, iinit, 0)

    nps = N_HIST // NS
    zoff = s * nps
    pltpu.sync_copy(hist.at[pl.ds(0, nps)], degsh.at[pl.ds(zoff, nps)])
    plsc.subcore_barrier()

    def chunk(k, carry):
        base = c * (E_PAD // NC) + s * DEG_EPT + k * C
        pltpu.sync_copy(dst_hbm.at[pl.ds(base, C)], dstv)
        pltpu.sync_copy(ew_hbm.at[pl.ds(base, C)], eww)
        for i in range(C // L):
            idx = dstv[pl.ds(i * L, L)]
            val = eww[pl.ds(i * L, L)]
            plsc.addupdate_scatter(hist, [idx], val)
        return carry

    lax.fori_loop(0, DEG_EPT // C, chunk, 0)

    def comb(j, carry):
        pltpu.sync_copy(hist.at[pl.ds(j * C, C)], degsh.at[id2d.at[j]], add=True)
        return carry

    lax.fori_loop(0, N_HIST // C, comb, 0)
    plsc.subcore_barrier()

    pltpu.sync_copy(degsh.at[pl.ds(zoff, nps)], hist.at[pl.ds(0, nps)])

    @pl.when(c == 0)
    def _():
        pltpu.sync_copy(hist.at[pl.ds(0, nps)], deg0_hbm.at[pl.ds(zoff, nps)])

    @pl.when(c == 1)
    def _():
        pltpu.sync_copy(hist.at[pl.ds(0, nps)], deg1_hbm.at[pl.ds(zoff, nps)])


_deg_kernel = functools.partial(
    pl.kernel,
    out_type=(jax.ShapeDtypeStruct((N_HIST,), jnp.float32),
              jax.ShapeDtypeStruct((N_HIST,), jnp.float32)),
    mesh=_mesh,
    scratch_types=[
        pltpu.VMEM((N_HIST,), jnp.float32),
        pltpu.VMEM((N_HIST // C, C), jnp.int32),
        pltpu.VMEM((C,), jnp.int32),
        pltpu.VMEM((C,), jnp.float32),
        pltpu.VMEM_SHARED((N_HIST,), jnp.float32),
    ],
    compiler_params=pltpu.CompilerParams(needs_layout_passes=False),
)(_deg_body)


def _agg_body(y_hbm, src_hbm, dst_hbm, ew_hbm, t_hbm,
              srcv, gidx, dstv, eww, rows, acc, sem):
    c = lax.axis_index("c")
    s = lax.axis_index("s")

    _zero_rows(rows, C, HALF)
    abase = s * 632
    for off in (0, C, 2 * C, 3 * C):
        pltpu.sync_copy(rows.at[pl.ds(0, C)], acc.at[pl.ds(abase + off, C)])

    @pl.when(s < NS - 1)
    def _():
        pltpu.sync_copy(rows.at[pl.ds(0, 120)], acc.at[pl.ds(abase + 4 * C, 120)])

    @pl.when(s == NS - 1)
    def _():
        pltpu.sync_copy(rows.at[pl.ds(0, 8)], acc.at[pl.ds(abase + 4 * C, 8)])

    plsc.subcore_barrier()

    coff = c * N

    def chunk(k, carry):
        base = s * EPT + k * C
        pltpu.sync_copy(src_hbm.at[pl.ds(base, C)], srcv)
        pltpu.sync_copy(dst_hbm.at[pl.ds(base, C)], dstv)
        pltpu.sync_copy(ew_hbm.at[pl.ds(base, C)], eww)
        for i in range(C // L):
            gidx[pl.ds(i * L, L)] = srcv[pl.ds(i * L, L)] + coff
        pltpu.async_copy(y_hbm.at[gidx], rows, sem).wait()

        def scale(e, carry2):
            sp = plsc.load_gather(eww, [jnp.full((L,), e, jnp.int32)])
            for j in range(HALF // L):
                rows[e, pl.ds(j * L, L)] = rows[e, pl.ds(j * L, L)] * sp
            return carry2

        lax.fori_loop(0, C, scale, 0)
        pltpu.sync_copy(rows, acc.at[dstv], add=True)
        return carry

    lax.fori_loop(0, EPT // C, chunk, 0)
    plsc.subcore_barrier()

    def drain(off, sz):
        pltpu.sync_copy(acc.at[pl.ds(abase + off, sz)], rows.at[pl.ds(0, sz)])
        pltpu.sync_copy(rows.at[pl.ds(0, sz)],
                        t_hbm.at[pl.ds(coff + abase + off, sz)])

    for off in (0, C, 2 * C, 3 * C):
        drain(off, C)

    @pl.when(s < NS - 1)
    def _():
        drain(4 * C, 120)

    @pl.when(s == NS - 1)
    def _():
        drain(4 * C, 8)


_agg_kernel = functools.partial(
    pl.kernel,
    out_type=jax.ShapeDtypeStruct((2 * N, HALF), jnp.float32),
    mesh=_mesh,
    scratch_types=[
        pltpu.VMEM((C,), jnp.int32),
        pltpu.VMEM((C,), jnp.int32),
        pltpu.VMEM((C,), jnp.int32),
        pltpu.VMEM((C,), jnp.float32),
        pltpu.VMEM((C, HALF), jnp.float32),
        pltpu.VMEM_SHARED((N, HALF), jnp.float32),
        pltpu.SemaphoreType.DMA,
    ],
    compiler_params=pltpu.CompilerParams(needs_layout_passes=False),
)(_agg_body)


def _dinv(d0, d1):
    deg = d0 + d1 + 1.0
    return jnp.where(deg > 0, lax.rsqrt(deg), 0.0)


def _prescale_body(x_ref, d0_ref, d1_ref, y_ref):
    y_ref[...] = _dinv(d0_ref[...], d1_ref[...]) * x_ref[...]


def _main_body(t1a_ref, t1b_ref, y1a_ref, y1b_ref, d0_ref, d1_ref,
               w1_ref, b1_ref, w2_ref, y2_ref):
    dinv = _dinv(d0_ref[...], d1_ref[...])
    agg = jnp.concatenate(
        [dinv * (t1a_ref[...] + y1a_ref[...]),
         dinv * (t1b_ref[...] + y1b_ref[...])], axis=1)
    h = jnp.maximum(
        jnp.dot(agg, w1_ref[...], preferred_element_type=jnp.float32)
        + b1_ref[...], 0.0)
    z = jnp.dot(h, w2_ref[...], preferred_element_type=jnp.float32)
    y2_ref[0] = _dinv(d0_ref[...], d1_ref[...]) * z[:, :HALF]
    y2_ref[1] = _dinv(d0_ref[...], d1_ref[...]) * z[:, HALF:]


def _final_body(t2_ref, y2_ref, d0_ref, d1_ref, b2_ref, out_ref):
    h = pl.program_id(1)
    dinv = _dinv(d0_ref[...], d1_ref[...])
    out_ref[...] = dinv * (t2_ref[...] + y2_ref[...]) + b2_ref[pl.ds(h, 1), :]


def kernel(x, edge_index, edge_weight, W1, b1, W2, b2):
    src = edge_index[0].astype(jnp.int32)
    dst = edge_index[1].astype(jnp.int32)
    ew = edge_weight.astype(jnp.float32)
    pad = E_PAD - E
    src_p = jnp.concatenate([src, jnp.zeros((pad,), jnp.int32)])
    dst_p = jnp.concatenate([dst, jnp.zeros((pad,), jnp.int32)])
    ew_p = jnp.concatenate([ew, jnp.zeros((pad,), jnp.float32)])

    deg0, deg1 = _deg_kernel(dst_p, ew_p)
    d0 = deg0[:N].reshape(N, 1)
    d1 = deg1[:N].reshape(N, 1)

    RB = 2000
    nrb = N // RB
    y1 = pl.pallas_call(
        _prescale_body,
        grid=(nrb, 2),
        in_specs=[
            pl.BlockSpec((RB, HALF), lambda i, h: (i, h)),
            pl.BlockSpec((RB, 1), lambda i, h: (i, 0)),
            pl.BlockSpec((RB, 1), lambda i, h: (i, 0)),
        ],
        out_specs=pl.BlockSpec((RB, HALF), lambda i, h: (i + h * nrb, 0)),
        out_shape=jax.ShapeDtypeStruct((2 * N, HALF), jnp.float32),
    )(x, d0, d1)

    t1 = _agg_kernel(y1, src_p, dst_p, ew_p)

    RM = 1000
    nrm = N // RM
    y2_3d = pl.pallas_call(
        _main_body,
        grid=(nrm,),
        in_specs=[
            pl.BlockSpec((RM, HALF), lambda i: (i, 0)),
            pl.BlockSpec((RM, HALF), lambda i: (i + nrm, 0)),
            pl.BlockSpec((RM, HALF), lambda i: (i, 0)),
            pl.BlockSpec((RM, HALF), lambda i: (i + nrm, 0)),
            pl.BlockSpec((RM, 1), lambda i: (i, 0)),
            pl.BlockSpec((RM, 1), lambda i: (i, 0)),
            pl.BlockSpec((D_IN, D_HID), lambda i: (0, 0)),
            pl.BlockSpec((1, D_HID), lambda i: (0, 0)),
            pl.BlockSpec((D_HID, D_OUT), lambda i: (0, 0)),
        ],
        out_specs=pl.BlockSpec((2, RM, HALF), lambda i: (0, i, 0)),
        out_shape=jax.ShapeDtypeStruct((2, N, HALF), jnp.float32),
    )(t1, t1, y1, y1, d0, d1, W1, b1.reshape(1, D_HID), W2)
    y2 = y2_3d.reshape(2 * N, HALF)

    t2 = _agg_kernel(y2, src_p, dst_p, ew_p)

    out = pl.pallas_call(
        _final_body,
        grid=(nrb, 2),
        in_specs=[
            pl.BlockSpec((RB, HALF), lambda i, h: (i + h * nrb, 0)),
            pl.BlockSpec((RB, HALF), lambda i, h: (i + h * nrb, 0)),
            pl.BlockSpec((RB, 1), lambda i, h: (i, 0)),
            pl.BlockSpec((RB, 1), lambda i, h: (i, 0)),
            pl.BlockSpec((2, HALF), lambda i, h: (0, 0)),
        ],
        out_specs=pl.BlockSpec((RB, HALF), lambda i, h: (i, h)),
        out_shape=jax.ShapeDtypeStruct((N, D_OUT), jnp.float32),
    )(t2, y2, d0, d1, b2.reshape(2, HALF))
    return out


# speedup vs baseline: 9.0233x; 1.5355x over previous
import functools

import jax
import jax.numpy as jnp
from jax import lax
from jax.experimental import pallas as pl
from jax.experimental.pallas import tpu as pltpu
from jax.experimental.pallas import tpu_sc as plsc

N = 10000
E = 160000
D_IN = 256
D_HID = 512
D_OUT = 256
HALF = 128
NC = 2
NS = 16
L = 16
C = 128
EPT = 10240
E_PAD = NS * EPT
NPT = N // NS
N_HIST = 10112
DEG_EPT = E_PAD // (NC * NS)

_mesh = plsc.VectorSubcoreMesh(
    core_axis_name="c", subcore_axis_name="s", num_cores=NC, num_subcores=NS)


def _zero_rows(buf, nrows, width):
    zz = jnp.zeros((L,), jnp.float32)

    def body(e, carry):
        for j in range(width // L):
            buf[e, pl.ds(j * L, L)] = zz
        return carry

    lax.fori_loop(0, nrows, body, 0)


def _deg_body(dst_hbm, ew_hbm, deg0_hbm, deg1_hbm,
              hist, id2d, dstv, eww, degsh):
    c = lax.axis_index("c")
    s = lax.axis_index("s")

    zz = jnp.zeros((L,), jnp.float32)

    def zinit(k, carry):
        hist[pl.ds(k * L, L)] = zz
        return carry

    lax.fori_loop(0, N_HIST // L, zinit, 0)

    def iinit(k, carry):
        base = k * L + lax.broadcasted_iota(jnp.int32, (L,), 0)
        j = k // (C // L)
        i = k % (C // L)
        id2d[j, pl.ds(i * L, L)] = base
        return carry

    lax.fori_loop(0, N_HIST // L, iinit, 0)

    nps = N_HIST // NS
    zoff = s * nps
    pltpu.sync_copy(hist.at[pl.ds(0, nps)], degsh.at[pl.ds(zoff, nps)])
    plsc.subcore_barrier()

    def chunk(k, carry):
        base = c * (E_PAD // NC) + s * DEG_EPT + k * C
        pltpu.sync_copy(dst_hbm.at[pl.ds(base, C)], dstv)
        pltpu.sync_copy(ew_hbm.at[pl.ds(base, C)], eww)
        for i in range(C // L):
            idx = dstv[pl.ds(i * L, L)]
            val = eww[pl.ds(i * L, L)]
            plsc.addupdate_scatter(hist, [idx], val)
        return carry

    lax.fori_loop(0, DEG_EPT // C, chunk, 0)

    def comb(j, carry):
        pltpu.sync_copy(hist.at[pl.ds(j * C, C)], degsh.at[id2d.at[j]], add=True)
        return carry

    lax.fori_loop(0, N_HIST // C, comb, 0)
    plsc.subcore_barrier()

    pltpu.sync_copy(degsh.at[pl.ds(zoff, nps)], hist.at[pl.ds(0, nps)])

    @pl.when(c == 0)
    def _():
        pltpu.sync_copy(hist.at[pl.ds(0, nps)], deg0_hbm.at[pl.ds(zoff, nps)])

    @pl.when(c == 1)
    def _():
        pltpu.sync_copy(hist.at[pl.ds(0, nps)], deg1_hbm.at[pl.ds(zoff, nps)])


_deg_kernel = functools.partial(
    pl.kernel,
    out_type=(jax.ShapeDtypeStruct((N_HIST,), jnp.float32),
              jax.ShapeDtypeStruct((N_HIST,), jnp.float32)),
    mesh=_mesh,
    scratch_types=[
        pltpu.VMEM((N_HIST,), jnp.float32),
        pltpu.VMEM((N_HIST // C, C), jnp.int32),
        pltpu.VMEM((C,), jnp.int32),
        pltpu.VMEM((C,), jnp.float32),
        pltpu.VMEM_SHARED((N_HIST,), jnp.float32),
    ],
    compiler_params=pltpu.CompilerParams(needs_layout_passes=False),
)(_deg_body)


def _agg_body(y_hbm, src2d_hbm, dst2d_hbm, ew2d_hbm, t_hbm,
              gidx2d, dstbuf, ewbuf, rows0, rows1,
              acc, gsem0, gsem1):
    c = lax.axis_index("c")
    s = lax.axis_index("s")
    NCH = EPT // C
    SCH = NCH // 2
    rbase = s * NCH
    coff = c * N

    _zero_rows(rows0, C, HALF)
    abase = s * 632
    for off in (0, C, 2 * C, 3 * C):
        pltpu.sync_copy(rows0.at[pl.ds(0, C)], acc.at[pl.ds(abase + off, C)])

    @pl.when(s < NS - 1)
    def _():
        pltpu.sync_copy(rows0.at[pl.ds(0, 120)], acc.at[pl.ds(abase + 4 * C, 120)])

    @pl.when(s == NS - 1)
    def _():
        pltpu.sync_copy(rows0.at[pl.ds(0, 8)], acc.at[pl.ds(abase + 4 * C, 8)])

    plsc.subcore_barrier()

    bufs = ((rows0, gsem0), (rows1, gsem1))
    for hh in range(2):
        srow = rbase + hh * SCH
        pltpu.sync_copy(src2d_hbm.at[pl.ds(srow, SCH)], gidx2d)
        pltpu.sync_copy(dst2d_hbm.at[pl.ds(srow, SCH)], dstbuf)
        pltpu.sync_copy(ew2d_hbm.at[pl.ds(srow, SCH)], ewbuf)

        def gset(r, carry):
            for i in range(C // L):
                gidx2d[r, pl.ds(i * L, L)] = gidx2d[r, pl.ds(i * L, L)] + coff
            return carry

        lax.fori_loop(0, SCH, gset, 0)

        for b in range(2):
            pltpu.async_copy(y_hbm.at[gidx2d.at[b]], bufs[b][0], bufs[b][1])

        def pair(p, carry):
            for b in range(2):
                k = p * 2 + b
                rb, gs = bufs[b]
                pltpu.make_async_copy(y_hbm.at[gidx2d.at[k]], rb, gs).wait()

                def scale(e, carry2):
                    sp = plsc.load_gather(
                        ewbuf, [jnp.full((L,), k, jnp.int32),
                                jnp.full((L,), e, jnp.int32)])
                    for j in range(HALF // L):
                        rb[e, pl.ds(j * L, L)] = rb[e, pl.ds(j * L, L)] * sp
                    return carry2

                lax.fori_loop(0, C, scale, 0)
                pltpu.sync_copy(rb, acc.at[dstbuf.at[k]], add=True)

                @pl.when(k + 2 < SCH)
                def _():
                    pltpu.async_copy(y_hbm.at[gidx2d.at[k + 2]], rb, gs)

            return carry

        lax.fori_loop(0, SCH // 2, pair, 0)

    plsc.subcore_barrier()

    def drain(off, sz):
        pltpu.sync_copy(acc.at[pl.ds(abase + off, sz)], rows0.at[pl.ds(0, sz)])
        pltpu.sync_copy(rows0.at[pl.ds(0, sz)],
                        t_hbm.at[pl.ds(coff + abase + off, sz)])

    for off in (0, C, 2 * C, 3 * C):
        drain(off, C)

    @pl.when(s < NS - 1)
    def _():
        drain(4 * C, 120)

    @pl.when(s == NS - 1)
    def _():
        drain(4 * C, 8)


_agg_kernel = functools.partial(
    pl.kernel,
    out_type=jax.ShapeDtypeStruct((2 * N, HALF), jnp.float32),
    mesh=_mesh,
    scratch_types=[
        pltpu.VMEM((EPT // C // 2, C), jnp.int32),
        pltpu.VMEM((EPT // C // 2, C), jnp.int32),
        pltpu.VMEM((EPT // C // 2, C), jnp.float32),
        pltpu.VMEM((C, HALF), jnp.float32),
        pltpu.VMEM((C, HALF), jnp.float32),
        pltpu.VMEM_SHARED((N, HALF), jnp.float32),
        pltpu.SemaphoreType.DMA,
        pltpu.SemaphoreType.DMA,
    ],
    compiler_params=pltpu.CompilerParams(needs_layout_passes=False),
)(_agg_body)


def _dinv(d0, d1):
    deg = d0 + d1 + 1.0
    return jnp.where(deg > 0, lax.rsqrt(deg), 0.0)


def _prescale_body(x_ref, d0_ref, d1_ref, y_ref):
    y_ref[...] = _dinv(d0_ref[...], d1_ref[...]) * x_ref[...]


def _main_body(t1a_ref, t1b_ref, y1a_ref, y1b_ref, d0_ref, d1_ref,
               w1_ref, b1_ref, w2_ref, y2_ref):
    dinv = _dinv(d0_ref[...], d1_ref[...])
    agg = jnp.concatenate(
        [dinv * (t1a_ref[...] + y1a_ref[...]),
         dinv * (t1b_ref[...] + y1b_ref[...])], axis=1)
    h = jnp.maximum(
        jnp.dot(agg, w1_ref[...], preferred_element_type=jnp.float32)
        + b1_ref[...], 0.0)
    z = jnp.dot(h, w2_ref[...], preferred_element_type=jnp.float32)
    y2_ref[0] = _dinv(d0_ref[...], d1_ref[...]) * z[:, :HALF]
    y2_ref[1] = _dinv(d0_ref[...], d1_ref[...]) * z[:, HALF:]


def _final_body(t2_ref, y2_ref, d0_ref, d1_ref, b2_ref, out_ref):
    h = pl.program_id(1)
    dinv = _dinv(d0_ref[...], d1_ref[...])
    out_ref[...] = dinv * (t2_ref[...] + y2_ref[...]) + b2_ref[pl.ds(h, 1), :]


def kernel(x, edge_index, edge_weight, W1, b1, W2, b2):
    src = edge_index[0].astype(jnp.int32)
    dst = edge_index[1].astype(jnp.int32)
    ew = edge_weight.astype(jnp.float32)
    pad = E_PAD - E
    src_p = jnp.concatenate([src, jnp.zeros((pad,), jnp.int32)])
    dst_p = jnp.concatenate([dst, jnp.zeros((pad,), jnp.int32)])
    ew_p = jnp.concatenate([ew, jnp.zeros((pad,), jnp.float32)])

    src2d = src_p.reshape(E_PAD // C, C)
    dst2d = dst_p.reshape(E_PAD // C, C)
    ew2d = ew_p.reshape(E_PAD // C, C)

    deg0, deg1 = _deg_kernel(dst_p, ew_p)
    d0 = deg0[:N].reshape(N, 1)
    d1 = deg1[:N].reshape(N, 1)

    RB = 2000
    nrb = N // RB
    y1 = pl.pallas_call(
        _prescale_body,
        grid=(nrb, 2),
        in_specs=[
            pl.BlockSpec((RB, HALF), lambda i, h: (i, h)),
            pl.BlockSpec((RB, 1), lambda i, h: (i, 0)),
            pl.BlockSpec((RB, 1), lambda i, h: (i, 0)),
        ],
        out_specs=pl.BlockSpec((RB, HALF), lambda i, h: (i + h * nrb, 0)),
        out_shape=jax.ShapeDtypeStruct((2 * N, HALF), jnp.float32),
    )(x, d0, d1)

    t1 = _agg_kernel(y1, src2d, dst2d, ew2d)

    RM = 1000
    nrm = N // RM
    y2_3d = pl.pallas_call(
        _main_body,
        grid=(nrm,),
        in_specs=[
            pl.BlockSpec((RM, HALF), lambda i: (i, 0)),
            pl.BlockSpec((RM, HALF), lambda i: (i + nrm, 0)),
            pl.BlockSpec((RM, HALF), lambda i: (i, 0)),
            pl.BlockSpec((RM, HALF), lambda i: (i + nrm, 0)),
            pl.BlockSpec((RM, 1), lambda i: (i, 0)),
            pl.BlockSpec((RM, 1), lambda i: (i, 0)),
            pl.BlockSpec((D_IN, D_HID), lambda i: (0, 0)),
            pl.BlockSpec((1, D_HID), lambda i: (0, 0)),
            pl.BlockSpec((D_HID, D_OUT), lambda i: (0, 0)),
        ],
        out_specs=pl.BlockSpec((2, RM, HALF), lambda i: (0, i, 0)),
        out_shape=jax.ShapeDtypeStruct((2, N, HALF), jnp.float32),
    )(t1, t1, y1, y1, d0, d1, W1, b1.reshape(1, D_HID), W2)
    y2 = y2_3d.reshape(2 * N, HALF)

    t2 = _agg_kernel(y2, src2d, dst2d, ew2d)

    out = pl.pallas_call(
        _final_body,
        grid=(nrb, 2),
        in_specs=[
            pl.BlockSpec((RB, HALF), lambda i, h: (i + h * nrb, 0)),
            pl.BlockSpec((RB, HALF), lambda i, h: (i + h * nrb, 0)),
            pl.BlockSpec((RB, 1), lambda i, h: (i, 0)),
            pl.BlockSpec((RB, 1), lambda i, h: (i, 0)),
            pl.BlockSpec((2, HALF), lambda i, h: (0, 0)),
        ],
        out_specs=pl.BlockSpec((RB, HALF), lambda i, h: (i, h)),
        out_shape=jax.ShapeDtypeStruct((N, D_OUT), jnp.float32),
    )(t2, y2, d0, d1, b2.reshape(2, HALF))
    return out


# speedup vs baseline: 9.0682x; 1.0050x over previous
import functools

import jax
import jax.numpy as jnp
from jax import lax
from jax.experimental import pallas as pl
from jax.experimental.pallas import tpu as pltpu
from jax.experimental.pallas import tpu_sc as plsc

N = 10000
E = 160000
D_IN = 256
D_HID = 512
D_OUT = 256
HALF = 128
NC = 2
NS = 16
L = 16
C = 128
EPT = 10240
E_PAD = NS * EPT
NPT = N // NS
N_HIST = 10112
DEG_EPT = E_PAD // (NC * NS)

_mesh = plsc.VectorSubcoreMesh(
    core_axis_name="c", subcore_axis_name="s", num_cores=NC, num_subcores=NS)


def _zero_rows(buf, nrows, width):
    zz = jnp.zeros((L,), jnp.float32)

    def body(e, carry):
        for j in range(width // L):
            buf[e, pl.ds(j * L, L)] = zz
        return carry

    lax.fori_loop(0, nrows, body, 0)


def _deg_body(dst_hbm, ew_hbm, deg0_hbm, deg1_hbm,
              hist, id2d, dstv, eww, degsh):
    c = lax.axis_index("c")
    s = lax.axis_index("s")

    zz = jnp.zeros((L,), jnp.float32)

    def zinit(k, carry):
        hist[pl.ds(k * L, L)] = zz
        return carry

    lax.fori_loop(0, N_HIST // L, zinit, 0)

    def iinit(k, carry):
        base = k * L + lax.broadcasted_iota(jnp.int32, (L,), 0)
        j = k // (C // L)
        i = k % (C // L)
        id2d[j, pl.ds(i * L, L)] = base
        return carry

    lax.fori_loop(0, N_HIST // L, iinit, 0)

    nps = N_HIST // NS
    zoff = s * nps
    pltpu.sync_copy(hist.at[pl.ds(0, nps)], degsh.at[pl.ds(zoff, nps)])
    plsc.subcore_barrier()

    def chunk(k, carry):
        base = c * (E_PAD // NC) + s * DEG_EPT + k * C
        pltpu.sync_copy(dst_hbm.at[pl.ds(base, C)], dstv)
        pltpu.sync_copy(ew_hbm.at[pl.ds(base, C)], eww)
        for i in range(C // L):
            idx = dstv[pl.ds(i * L, L)]
            val = eww[pl.ds(i * L, L)]
            plsc.addupdate_scatter(hist, [idx], val)
        return carry

    lax.fori_loop(0, DEG_EPT // C, chunk, 0)

    def comb(j, carry):
        pltpu.sync_copy(hist.at[pl.ds(j * C, C)], degsh.at[id2d.at[j]], add=True)
        return carry

    lax.fori_loop(0, N_HIST // C, comb, 0)
    plsc.subcore_barrier()

    pltpu.sync_copy(degsh.at[pl.ds(zoff, nps)], hist.at[pl.ds(0, nps)])

    @pl.when(c == 0)
    def _():
        pltpu.sync_copy(hist.at[pl.ds(0, nps)], deg0_hbm.at[pl.ds(zoff, nps)])

    @pl.when(c == 1)
    def _():
        pltpu.sync_copy(hist.at[pl.ds(0, nps)], deg1_hbm.at[pl.ds(zoff, nps)])


_deg_kernel = functools.partial(
    pl.kernel,
    out_type=(jax.ShapeDtypeStruct((N_HIST,), jnp.float32),
              jax.ShapeDtypeStruct((N_HIST,), jnp.float32)),
    mesh=_mesh,
    scratch_types=[
        pltpu.VMEM((N_HIST,), jnp.float32),
        pltpu.VMEM((N_HIST // C, C), jnp.int32),
        pltpu.VMEM((C,), jnp.int32),
        pltpu.VMEM((C,), jnp.float32),
        pltpu.VMEM_SHARED((N_HIST,), jnp.float32),
    ],
    compiler_params=pltpu.CompilerParams(needs_layout_passes=False),
)(_deg_body)


def _agg_body(y_hbm, src2d_hbm, dst2d_hbm, ew2d_hbm, t_hbm,
              gidx2d, dstbuf, ewbuf, rows0, rows1,
              acc, gsem0, gsem1):
    c = lax.axis_index("c")
    s = lax.axis_index("s")
    NCH = EPT // C
    SCH = NCH // 2
    rbase = s * NCH
    coff = c * N

    _zero_rows(rows0, C, HALF)
    abase = s * 632
    for off in (0, C, 2 * C, 3 * C):
        pltpu.sync_copy(rows0.at[pl.ds(0, C)], acc.at[pl.ds(abase + off, C)])

    @pl.when(s < NS - 1)
    def _():
        pltpu.sync_copy(rows0.at[pl.ds(0, 120)], acc.at[pl.ds(abase + 4 * C, 120)])

    @pl.when(s == NS - 1)
    def _():
        pltpu.sync_copy(rows0.at[pl.ds(0, 8)], acc.at[pl.ds(abase + 4 * C, 8)])

    plsc.subcore_barrier()

    bufs = ((rows0, gsem0), (rows1, gsem1))
    for hh in range(2):
        srow = rbase + hh * SCH
        pltpu.sync_copy(src2d_hbm.at[pl.ds(srow, SCH)], gidx2d)
        pltpu.sync_copy(dst2d_hbm.at[pl.ds(srow, SCH)], dstbuf)
        pltpu.sync_copy(ew2d_hbm.at[pl.ds(srow, SCH)], ewbuf)

        def gset(r, carry):
            for i in range(C // L):
                gidx2d[r, pl.ds(i * L, L)] = gidx2d[r, pl.ds(i * L, L)] + coff
            return carry

        lax.fori_loop(0, SCH, gset, 0)

        def fire(k, rb, gs):
            for q in range(4):
                pltpu.async_copy(
                    y_hbm.at[gidx2d.at[k, pl.ds(q * 32, 32)]],
                    rb.at[pl.ds(q * 32, 32)], gs)

        def drain(k, rb, gs):
            for q in range(4):
                pltpu.make_async_copy(
                    y_hbm.at[gidx2d.at[k, pl.ds(q * 32, 32)]],
                    rb.at[pl.ds(q * 32, 32)], gs).wait()

        for b in range(2):
            fire(b, bufs[b][0], bufs[b][1])

        def pair(p, carry):
            for b in range(2):
                k = p * 2 + b
                rb, gs = bufs[b]
                drain(k, rb, gs)

                def scale(e, carry2):
                    sp = plsc.load_gather(
                        ewbuf, [jnp.full((L,), k, jnp.int32),
                                jnp.full((L,), e, jnp.int32)])
                    for j in range(HALF // L):
                        rb[e, pl.ds(j * L, L)] = rb[e, pl.ds(j * L, L)] * sp
                    return carry2

                lax.fori_loop(0, C, scale, 0)
                pltpu.sync_copy(rb, acc.at[dstbuf.at[k]], add=True)

                @pl.when(k + 2 < SCH)
                def _():
                    fire(k + 2, rb, gs)

            return carry

        lax.fori_loop(0, SCH // 2, pair, 0)

    plsc.subcore_barrier()

    def drain(off, sz):
        pltpu.sync_copy(acc.at[pl.ds(abase + off, sz)], rows0.at[pl.ds(0, sz)])
        pltpu.sync_copy(rows0.at[pl.ds(0, sz)],
                        t_hbm.at[pl.ds(coff + abase + off, sz)])

    for off in (0, C, 2 * C, 3 * C):
        drain(off, C)

    @pl.when(s < NS - 1)
    def _():
        drain(4 * C, 120)

    @pl.when(s == NS - 1)
    def _():
        drain(4 * C, 8)


_agg_kernel = functools.partial(
    pl.kernel,
    out_type=jax.ShapeDtypeStruct((2 * N, HALF), jnp.float32),
    mesh=_mesh,
    scratch_types=[
        pltpu.VMEM((EPT // C // 2, C), jnp.int32),
        pltpu.VMEM((EPT // C // 2, C), jnp.int32),
        pltpu.VMEM((EPT // C // 2, C), jnp.float32),
        pltpu.VMEM((C, HALF), jnp.float32),
        pltpu.VMEM((C, HALF), jnp.float32),
        pltpu.VMEM_SHARED((N, HALF), jnp.float32),
        pltpu.SemaphoreType.DMA,
        pltpu.SemaphoreType.DMA,
    ],
    compiler_params=pltpu.CompilerParams(needs_layout_passes=False),
)(_agg_body)


def _dinv(d0, d1):
    deg = d0 + d1 + 1.0
    return jnp.where(deg > 0, lax.rsqrt(deg), 0.0)


def _prescale_body(x_ref, d0_ref, d1_ref, y_ref):
    y_ref[...] = _dinv(d0_ref[...], d1_ref[...]) * x_ref[...]


def _main_body(t1a_ref, t1b_ref, y1a_ref, y1b_ref, d0_ref, d1_ref,
               w1_ref, b1_ref, w2_ref, y2_ref):
    dinv = _dinv(d0_ref[...], d1_ref[...])
    agg = jnp.concatenate(
        [dinv * (t1a_ref[...] + y1a_ref[...]),
         dinv * (t1b_ref[...] + y1b_ref[...])], axis=1)
    h = jnp.maximum(
        jnp.dot(agg, w1_ref[...], preferred_element_type=jnp.float32)
        + b1_ref[...], 0.0)
    z = jnp.dot(h, w2_ref[...], preferred_element_type=jnp.float32)
    y2_ref[0] = _dinv(d0_ref[...], d1_ref[...]) * z[:, :HALF]
    y2_ref[1] = _dinv(d0_ref[...], d1_ref[...]) * z[:, HALF:]


def _final_body(t2_ref, y2_ref, d0_ref, d1_ref, b2_ref, out_ref):
    h = pl.program_id(1)
    dinv = _dinv(d0_ref[...], d1_ref[...])
    out_ref[...] = dinv * (t2_ref[...] + y2_ref[...]) + b2_ref[pl.ds(h, 1), :]


def kernel(x, edge_index, edge_weight, W1, b1, W2, b2):
    src = edge_index[0].astype(jnp.int32)
    dst = edge_index[1].astype(jnp.int32)
    ew = edge_weight.astype(jnp.float32)
    pad = E_PAD - E
    src_p = jnp.concatenate([src, jnp.zeros((pad,), jnp.int32)])
    dst_p = jnp.concatenate([dst, jnp.zeros((pad,), jnp.int32)])
    ew_p = jnp.concatenate([ew, jnp.zeros((pad,), jnp.float32)])

    src2d = src_p.reshape(E_PAD // C, C)
    dst2d = dst_p.reshape(E_PAD // C, C)
    ew2d = ew_p.reshape(E_PAD // C, C)

    deg0, deg1 = _deg_kernel(dst_p, ew_p)
    d0 = deg0[:N].reshape(N, 1)
    d1 = deg1[:N].reshape(N, 1)

    RB = 2000
    nrb = N // RB
    y1 = pl.pallas_call(
        _prescale_body,
        grid=(nrb, 2),
        in_specs=[
            pl.BlockSpec((RB, HALF), lambda i, h: (i, h)),
            pl.BlockSpec((RB, 1), lambda i, h: (i, 0)),
            pl.BlockSpec((RB, 1), lambda i, h: (i, 0)),
        ],
        out_specs=pl.BlockSpec((RB, HALF), lambda i, h: (i + h * nrb, 0)),
        out_shape=jax.ShapeDtypeStruct((2 * N, HALF), jnp.float32),
    )(x, d0, d1)

    t1 = _agg_kernel(y1, src2d, dst2d, ew2d)

    RM = 1000
    nrm = N // RM
    y2_3d = pl.pallas_call(
        _main_body,
        grid=(nrm,),
        in_specs=[
            pl.BlockSpec((RM, HALF), lambda i: (i, 0)),
            pl.BlockSpec((RM, HALF), lambda i: (i + nrm, 0)),
            pl.BlockSpec((RM, HALF), lambda i: (i, 0)),
            pl.BlockSpec((RM, HALF), lambda i: (i + nrm, 0)),
            pl.BlockSpec((RM, 1), lambda i: (i, 0)),
            pl.BlockSpec((RM, 1), lambda i: (i, 0)),
            pl.BlockSpec((D_IN, D_HID), lambda i: (0, 0)),
            pl.BlockSpec((1, D_HID), lambda i: (0, 0)),
            pl.BlockSpec((D_HID, D_OUT), lambda i: (0, 0)),
        ],
        out_specs=pl.BlockSpec((2, RM, HALF), lambda i: (0, i, 0)),
        out_shape=jax.ShapeDtypeStruct((2, N, HALF), jnp.float32),
    )(t1, t1, y1, y1, d0, d1, W1, b1.reshape(1, D_HID), W2)
    y2 = y2_3d.reshape(2 * N, HALF)

    t2 = _agg_kernel(y2, src2d, dst2d, ew2d)

    out = pl.pallas_call(
        _final_body,
        grid=(nrb, 2),
        in_specs=[
            pl.BlockSpec((RB, HALF), lambda i, h: (i + h * nrb, 0)),
            pl.BlockSpec((RB, HALF), lambda i, h: (i + h * nrb, 0)),
            pl.BlockSpec((RB, 1), lambda i, h: (i, 0)),
            pl.BlockSpec((RB, 1), lambda i, h: (i, 0)),
            pl.BlockSpec((2, HALF), lambda i, h: (0, 0)),
        ],
        out_specs=pl.BlockSpec((RB, HALF), lambda i, h: (i, h)),
        out_shape=jax.ShapeDtypeStruct((N, D_OUT), jnp.float32),
    )(t2, y2, d0, d1, b2.reshape(2, HALF))
    return out
